# baseline (speedup 1.0000x reference)
import functools

import jax
import jax.numpy as jnp
from jax import lax
from jax.experimental import pallas as pl
from jax.experimental.pallas import tpu as pltpu
from jax.experimental.pallas import tpu_sc as plsc

DIM = 64
BATCH = 16384
NW = 32
BPW = BATCH // NW
NCHUNK = BPW // 128
NGROUP = BPW // 16


def _sc_body(users2, pos2, neg2, uemb, iemb, x_out, reg_out,
             idx_u, idx_p, idx_n, rows_u, rows_p, rows_n, x_v, reg_v, sem):
    wid = lax.axis_index("s") * 2 + lax.axis_index("c")
    base = wid * BPW

    pltpu.sync_copy(users2.at[pl.ds(wid * NCHUNK, NCHUNK)], idx_u)
    pltpu.sync_copy(pos2.at[pl.ds(wid * NCHUNK, NCHUNK)], idx_p)
    pltpu.sync_copy(neg2.at[pl.ds(wid * NCHUNK, NCHUNK)], idx_n)

    copies = []
    for tbl, idxv, rowsv in ((uemb, idx_u, rows_u),
                             (iemb, idx_p, rows_p),
                             (iemb, idx_n, rows_n)):
        for j in range(NCHUNK):
            copies.append(pltpu.async_copy(
                tbl.at[idxv.at[j]], rowsv.at[pl.ds(j * 128, 128)], sem))
    for c in copies:
        c.wait()

    lane = lax.iota(jnp.int32, 16)

    def group(g, racc):
        rows16 = g * 16 + lane
        sp = jnp.zeros((16,), jnp.float32)
        sn = jnp.zeros((16,), jnp.float32)
        for d in range(DIM):
            cols = jnp.full((16,), d, jnp.int32)
            u = plsc.load_gather(rows_u, [rows16, cols])
            p = plsc.load_gather(rows_p, [rows16, cols])
            n = plsc.load_gather(rows_n, [rows16, cols])
            sp = sp + u * p
            sn = sn + u * n
            racc = racc + (u * u + p * p + n * n)
        x_v[pl.ds(g * 16, 16)] = sp - sn
        return racc

    racc = lax.fori_loop(0, NGROUP, group, jnp.zeros((16,), jnp.float32))
    reg_v[...] = racc

    pltpu.sync_copy(x_v, x_out.at[pl.ds(base, BPW)])
    pltpu.sync_copy(reg_v, reg_out.at[wid])


_sc_kernel = functools.partial(
    pl.kernel,
    out_type=(
        jax.ShapeDtypeStruct((BATCH,), jnp.float32),
        jax.ShapeDtypeStruct((NW, 16), jnp.float32),
    ),
    mesh=plsc.VectorSubcoreMesh(core_axis_name="c", subcore_axis_name="s"),
    scratch_types=[
        pltpu.VMEM((NCHUNK, 128), jnp.int32),
        pltpu.VMEM((NCHUNK, 128), jnp.int32),
        pltpu.VMEM((NCHUNK, 128), jnp.int32),
        pltpu.VMEM((BPW, DIM), jnp.float32),
        pltpu.VMEM((BPW, DIM), jnp.float32),
        pltpu.VMEM((BPW, DIM), jnp.float32),
        pltpu.VMEM((BPW,), jnp.float32),
        pltpu.VMEM((16,), jnp.float32),
        pltpu.SemaphoreType.DMA,
    ],
    compiler_params=pltpu.CompilerParams(needs_layout_passes=False,
                                         use_tc_tiling_on_sc=False),
)(_sc_body)


def _tc_body(x_ref, reg_ref, rank_ref, regl_ref):
    x = x_ref[...]
    t = -x
    sp = jnp.maximum(t, 0.0) + jnp.log1p(jnp.exp(-jnp.abs(t)))
    rank_ref[0, 0] = jnp.sum(sp) * (1.0 / BATCH)
    regl_ref[0, 0] = jnp.sum(reg_ref[...]) * (1.0 / BATCH)


_tc_kernel = pl.pallas_call(
    _tc_body,
    out_shape=(
        jax.ShapeDtypeStruct((1, 1), jnp.float32),
        jax.ShapeDtypeStruct((1, 1), jnp.float32),
    ),
    in_specs=[
        pl.BlockSpec(memory_space=pltpu.VMEM),
        pl.BlockSpec(memory_space=pltpu.VMEM),
    ],
    out_specs=(
        pl.BlockSpec(memory_space=pltpu.SMEM),
        pl.BlockSpec(memory_space=pltpu.SMEM),
    ),
)


@jax.jit
def kernel(users, pos_items, neg_items, user_emb, item_emb):
    users2 = users.astype(jnp.int32).reshape(128, 128)
    pos2 = pos_items.astype(jnp.int32).reshape(128, 128)
    neg2 = neg_items.astype(jnp.int32).reshape(128, 128)
    x, reg_part = _sc_kernel(users2, pos2, neg2, user_emb, item_emb)
    rank, regl = _tc_kernel(x.reshape(128, 128), reg_part.reshape(4, 128))
    return (rank[0, 0], regl[0, 0])


# speedup vs baseline: 1.0602x; 1.0602x over previous
import functools

import jax
import jax.numpy as jnp
from jax import lax
from jax.experimental import pallas as pl
from jax.experimental.pallas import tpu as pltpu
from jax.experimental.pallas import tpu_sc as plsc

DIM = 64
PADW = 128
BATCH = 16384
NW = 32
BPW = BATCH // NW
NCHUNK = BPW // 128
HALF = BPW // 2


def _sc_body(users2, pos2, neg2, uemb, iemb, x_out, reg_out,
             idx_u, idx_p, idx_n, rows_u, rows_p, rows_n, x_v, reg_v, sem):
    wid = lax.axis_index("s") * 2 + lax.axis_index("c")
    base = wid * BPW

    pltpu.sync_copy(users2.at[pl.ds(wid * NCHUNK, NCHUNK)], idx_u)
    pltpu.sync_copy(pos2.at[pl.ds(wid * NCHUNK, NCHUNK)], idx_p)
    pltpu.sync_copy(neg2.at[pl.ds(wid * NCHUNK, NCHUNK)], idx_n)

    lane = lax.iota(jnp.int32, 16)
    racc = jnp.zeros((16,), jnp.float32)

    for h in range(2):
        copies = []
        for tbl, idxv, rowsv in ((uemb, idx_u, rows_u),
                                 (iemb, idx_p, rows_p),
                                 (iemb, idx_n, rows_n)):
            for j in range(2):
                copies.append(pltpu.async_copy(
                    tbl.at[idxv.at[2 * h + j]],
                    rowsv.at[pl.ds(j * 128, 128)], sem))
        for c in copies:
            c.wait()

        def group(g, racc):
            rows16 = g * 16 + lane
            sp = jnp.zeros((16,), jnp.float32)
            sn = jnp.zeros((16,), jnp.float32)
            for d in range(DIM):
                cols = jnp.full((16,), d, jnp.int32)
                u = plsc.load_gather(rows_u, [rows16, cols])
                p = plsc.load_gather(rows_p, [rows16, cols])
                n = plsc.load_gather(rows_n, [rows16, cols])
                sp = sp + u * p
                sn = sn + u * n
                racc = racc + (u * u + p * p + n * n)
            x_v[pl.ds(h * HALF + g * 16, 16)] = sp - sn
            return racc

        racc = lax.fori_loop(0, HALF // 16, group, racc)

    reg_v[...] = racc
    pltpu.sync_copy(x_v, x_out.at[pl.ds(base, BPW)])
    pltpu.sync_copy(reg_v, reg_out.at[wid])


_sc_kernel = functools.partial(
    pl.kernel,
    out_type=(
        jax.ShapeDtypeStruct((BATCH,), jnp.float32),
        jax.ShapeDtypeStruct((NW, 16), jnp.float32),
    ),
    mesh=plsc.VectorSubcoreMesh(core_axis_name="c", subcore_axis_name="s"),
    scratch_types=[
        pltpu.VMEM((NCHUNK, 128), jnp.int32),
        pltpu.VMEM((NCHUNK, 128), jnp.int32),
        pltpu.VMEM((NCHUNK, 128), jnp.int32),
        pltpu.VMEM((HALF, PADW), jnp.float32),
        pltpu.VMEM((HALF, PADW), jnp.float32),
        pltpu.VMEM((HALF, PADW), jnp.float32),
        pltpu.VMEM((BPW,), jnp.float32),
        pltpu.VMEM((16,), jnp.float32),
        pltpu.SemaphoreType.DMA,
    ],
    compiler_params=pltpu.CompilerParams(needs_layout_passes=False,
                                         use_tc_tiling_on_sc=False),
)(_sc_body)


def _tc_body(x_ref, reg_ref, rank_ref, regl_ref):
    x = x_ref[...]
    t = -x
    sp = jnp.maximum(t, 0.0) + jnp.log1p(jnp.exp(-jnp.abs(t)))
    rank_ref[0, 0] = jnp.sum(sp) * (1.0 / BATCH)
    regl_ref[0, 0] = jnp.sum(reg_ref[...]) * (1.0 / BATCH)


_tc_kernel = pl.pallas_call(
    _tc_body,
    out_shape=(
        jax.ShapeDtypeStruct((1, 1), jnp.float32),
        jax.ShapeDtypeStruct((1, 1), jnp.float32),
    ),
    in_specs=[
        pl.BlockSpec(memory_space=pltpu.VMEM),
        pl.BlockSpec(memory_space=pltpu.VMEM),
    ],
    out_specs=(
        pl.BlockSpec(memory_space=pltpu.SMEM),
        pl.BlockSpec(memory_space=pltpu.SMEM),
    ),
)


@jax.jit
def kernel(users, pos_items, neg_items, user_emb, item_emb):
    users2 = users.astype(jnp.int32).reshape(128, 128)
    pos2 = pos_items.astype(jnp.int32).reshape(128, 128)
    neg2 = neg_items.astype(jnp.int32).reshape(128, 128)
    upad = jnp.pad(user_emb, ((0, 0), (0, PADW - DIM)))
    ipad = jnp.pad(item_emb, ((0, 0), (0, PADW - DIM)))
    x, reg_part = _sc_kernel(users2, pos2, neg2, upad, ipad)
    rank, regl = _tc_kernel(x.reshape(128, 128), reg_part.reshape(4, 128))
    return (rank[0, 0], regl[0, 0])


# speedup vs baseline: 2.1917x; 2.0672x over previous
import functools

import jax
import jax.numpy as jnp
from jax import lax
from jax.experimental import pallas as pl
from jax.experimental.pallas import tpu as pltpu
from jax.experimental.pallas import tpu_sc as plsc

DIM = 64
BATCH = 16384
NW = 32
BPW = BATCH // NW
NGROUP = BPW // 16
NPAIR = NGROUP // 2
NBLK = 125000


def _sc_body(users3, pos3, neg3, uemb3, iemb3, x_out, reg_out,
             vid_u, vid_p, vid_n,
             su0, su1, sp0, sp1, sn0, sn1,
             x_v, reg_v, sem0, sem1):
    wid = lax.axis_index("s") * 2 + lax.axis_index("c")
    lane = lax.iota(jnp.int32, 16)
    zero16i = jnp.zeros((16,), jnp.int32)

    for src, vdst in ((users3, vid_u), (pos3, vid_p), (neg3, vid_n)):
        pltpu.sync_copy(src.at[wid], vdst)

    sems = (sem0, sem1)
    tabs = ((uemb3, vid_u, (su0, su1)),
            (iemb3, vid_p, (sp0, sp1)),
            (iemb3, vid_n, (sn0, sn1)))

    def gidx(g, vid):
        irow = jnp.full((16,), 0, jnp.int32) + lax.div(g * 16, 128)
        icol = lax.rem(g * 16, 128) + lane
        return plsc.load_gather(vid, [irow, icol]).astype(jnp.int32)

    def fire(g, buf):
        s = sems[buf]
        for tbl3, vid, stages in tabs:
            bv = lax.div(gidx(g, vid), 8)
            for i in range(16):
                pltpu.async_copy(tbl3.at[bv[i]], stages[buf].at[i], s)

    def drain(buf):
        s = sems[buf]
        for tbl3, vid, stages in tabs:
            for i in range(16):
                pltpu.make_async_copy(tbl3.at[0], stages[buf].at[0], s).wait()

    def compute(g, buf, racc):
        svs = []
        for tbl3, vid, stages in tabs:
            iv = gidx(g, vid)
            svs.append(lax.rem(iv, 8))
        svu, svp, svn = svs
        sp = jnp.zeros((16,), jnp.float32)
        sn = jnp.zeros((16,), jnp.float32)
        for d in range(DIM):
            cd = jnp.full((16,), d, jnp.int32)
            u = plsc.load_gather(su0 if buf == 0 else su1, [lane, svu, cd])
            p = plsc.load_gather(sp0 if buf == 0 else sp1, [lane, svp, cd])
            n = plsc.load_gather(sn0 if buf == 0 else sn1, [lane, svn, cd])
            sp = sp + u * p
            sn = sn + u * n
            racc = racc + (u * u + p * p + n * n)
        xrow = jnp.full((16,), 0, jnp.int32) + lax.div(g, 8)
        xcol = lax.rem(g, 8) * 16 + lane
        plsc.store_scatter(x_v, [xrow, xcol], sp - sn)
        return racc

    fire(0, 0)

    def pair(gg, racc):
        g0 = 2 * gg
        fire(g0 + 1, 1)
        drain(0)
        racc = compute(g0, 0, racc)

        @pl.when(gg < NPAIR - 1)
        def _():
            fire(g0 + 2, 0)

        drain(1)
        racc = compute(g0 + 1, 1, racc)
        return racc

    racc = lax.fori_loop(0, NPAIR, pair, jnp.zeros((16,), jnp.float32))

    plsc.store_scatter(reg_v, [zero16i, lane], racc)
    for k in range(1, 8):
        plsc.store_scatter(reg_v, [zero16i, k * 16 + lane],
                           jnp.zeros((16,), jnp.float32))

    pltpu.sync_copy(x_v, x_out.at[wid])
    pltpu.sync_copy(reg_v, reg_out.at[wid])


_sc_kernel = functools.partial(
    pl.kernel,
    out_type=(
        jax.ShapeDtypeStruct((NW, 4, 128), jnp.float32),
        jax.ShapeDtypeStruct((NW, 1, 128), jnp.float32),
    ),
    mesh=plsc.VectorSubcoreMesh(core_axis_name="c", subcore_axis_name="s"),
    scratch_types=[
        pltpu.VMEM((4, 128), jnp.float32),
        pltpu.VMEM((4, 128), jnp.float32),
        pltpu.VMEM((4, 128), jnp.float32),
        pltpu.VMEM((16, 8, DIM), jnp.float32),
        pltpu.VMEM((16, 8, DIM), jnp.float32),
        pltpu.VMEM((16, 8, DIM), jnp.float32),
        pltpu.VMEM((16, 8, DIM), jnp.float32),
        pltpu.VMEM((16, 8, DIM), jnp.float32),
        pltpu.VMEM((16, 8, DIM), jnp.float32),
        pltpu.VMEM((4, 128), jnp.float32),
        pltpu.VMEM((1, 128), jnp.float32),
        pltpu.SemaphoreType.DMA,
        pltpu.SemaphoreType.DMA,
    ],
    compiler_params=pltpu.CompilerParams(needs_layout_passes=False,
                                         use_tc_tiling_on_sc=True),
)(_sc_body)


def _tc_body(x_ref, reg_ref, rank_ref, regl_ref):
    x = x_ref[...]
    t = -x
    sp = jnp.maximum(t, 0.0) + jnp.log1p(jnp.exp(-jnp.abs(t)))
    rank_ref[0, 0] = jnp.sum(sp) * (1.0 / BATCH)
    regl_ref[0, 0] = jnp.sum(reg_ref[...]) * (1.0 / BATCH)


_tc_kernel = pl.pallas_call(
    _tc_body,
    out_shape=(
        jax.ShapeDtypeStruct((1, 1), jnp.float32),
        jax.ShapeDtypeStruct((1, 1), jnp.float32),
    ),
    in_specs=[
        pl.BlockSpec(memory_space=pltpu.VMEM),
        pl.BlockSpec(memory_space=pltpu.VMEM),
    ],
    out_specs=(
        pl.BlockSpec(memory_space=pltpu.SMEM),
        pl.BlockSpec(memory_space=pltpu.SMEM),
    ),
)


@jax.jit
def kernel(users, pos_items, neg_items, user_emb, item_emb):
    users3 = users.astype(jnp.float32).reshape(NW, 4, 128)
    pos3 = pos_items.astype(jnp.float32).reshape(NW, 4, 128)
    neg3 = neg_items.astype(jnp.float32).reshape(NW, 4, 128)
    uemb3 = user_emb[:NBLK * 8].reshape(NBLK, 8, DIM)
    iemb3 = item_emb[:NBLK * 8].reshape(NBLK, 8, DIM)
    x, reg_part = _sc_kernel(users3, pos3, neg3, uemb3, iemb3)
    rank, regl = _tc_kernel(x.reshape(128, 128), reg_part.reshape(32, 128))
    return (rank[0, 0], regl[0, 0])


# speedup vs baseline: 2.1945x; 1.0012x over previous
import functools

import jax
import jax.numpy as jnp
from jax import lax
from jax.experimental import pallas as pl
from jax.experimental.pallas import tpu as pltpu
from jax.experimental.pallas import tpu_sc as plsc

DIM = 64
BATCH = 16384
NW = 32
BPW = BATCH // NW
NGROUP = BPW // 16
NPAIR = NGROUP // 2
NBLK = 125000


def _sc_body(users3, pos3, neg3, uemb3, iemb3, x_out, reg_out,
             vid_u, vid_p, vid_n,
             su0, su1, sp0, sp1, sn0, sn1,
             x_v, reg_v, sem0, sem1):
    wid = lax.axis_index("s") * 2 + lax.axis_index("c")
    lane = lax.iota(jnp.int32, 16)
    zero16i = jnp.zeros((16,), jnp.int32)

    for src, vdst in ((users3, vid_u), (pos3, vid_p), (neg3, vid_n)):
        pltpu.sync_copy(src.at[wid], vdst)

    sems = (sem0, sem1)
    tabs = ((uemb3, vid_u, (su0, su1)),
            (iemb3, vid_p, (sp0, sp1)),
            (iemb3, vid_n, (sn0, sn1)))

    def gidx(g, vid):
        irow = jnp.full((16,), 0, jnp.int32) + lax.div(g * 16, 128)
        icol = lax.rem(g * 16, 128) + lane
        return plsc.load_gather(vid, [irow, icol]).astype(jnp.int32)

    def fire(g, buf):
        s = sems[buf]
        for tbl3, vid, stages in tabs:
            bv = lax.div(gidx(g, vid), 8)
            for i in range(16):
                pltpu.async_copy(tbl3.at[bv[i]], stages[buf].at[i], s)

    def drain(buf):
        s = sems[buf]
        for tbl3, vid, stages in tabs:
            pltpu.make_async_copy(tbl3.at[pl.ds(0, 16)], stages[buf], s).wait()

    def compute(g, buf, racc):
        svs = []
        for tbl3, vid, stages in tabs:
            iv = gidx(g, vid)
            svs.append(lax.rem(iv, 8))
        svu, svp, svn = svs
        sp = jnp.zeros((16,), jnp.float32)
        sn = jnp.zeros((16,), jnp.float32)
        for d in range(DIM):
            cd = jnp.full((16,), d, jnp.int32)
            u = plsc.load_gather(su0 if buf == 0 else su1, [lane, svu, cd])
            p = plsc.load_gather(sp0 if buf == 0 else sp1, [lane, svp, cd])
            n = plsc.load_gather(sn0 if buf == 0 else sn1, [lane, svn, cd])
            sp = sp + u * p
            sn = sn + u * n
            racc = racc + (u * u + p * p + n * n)
        xrow = jnp.full((16,), 0, jnp.int32) + lax.div(g, 8)
        xcol = lax.rem(g, 8) * 16 + lane
        plsc.store_scatter(x_v, [xrow, xcol], sp - sn)
        return racc

    fire(0, 0)

    def pair(gg, racc):
        g0 = 2 * gg
        fire(g0 + 1, 1)
        drain(0)
        racc = compute(g0, 0, racc)

        @pl.when(gg < NPAIR - 1)
        def _():
            fire(g0 + 2, 0)

        drain(1)
        racc = compute(g0 + 1, 1, racc)
        return racc

    racc = lax.fori_loop(0, NPAIR, pair, jnp.zeros((16,), jnp.float32))

    plsc.store_scatter(reg_v, [zero16i, lane], racc)
    for k in range(1, 8):
        plsc.store_scatter(reg_v, [zero16i, k * 16 + lane],
                           jnp.zeros((16,), jnp.float32))

    pltpu.sync_copy(x_v, x_out.at[wid])
    pltpu.sync_copy(reg_v, reg_out.at[wid])


_sc_kernel = functools.partial(
    pl.kernel,
    out_type=(
        jax.ShapeDtypeStruct((NW, 4, 128), jnp.float32),
        jax.ShapeDtypeStruct((NW, 1, 128), jnp.float32),
    ),
    mesh=plsc.VectorSubcoreMesh(core_axis_name="c", subcore_axis_name="s"),
    scratch_types=[
        pltpu.VMEM((4, 128), jnp.float32),
        pltpu.VMEM((4, 128), jnp.float32),
        pltpu.VMEM((4, 128), jnp.float32),
        pltpu.VMEM((16, 8, DIM), jnp.float32),
        pltpu.VMEM((16, 8, DIM), jnp.float32),
        pltpu.VMEM((16, 8, DIM), jnp.float32),
        pltpu.VMEM((16, 8, DIM), jnp.float32),
        pltpu.VMEM((16, 8, DIM), jnp.float32),
        pltpu.VMEM((16, 8, DIM), jnp.float32),
        pltpu.VMEM((4, 128), jnp.float32),
        pltpu.VMEM((1, 128), jnp.float32),
        pltpu.SemaphoreType.DMA,
        pltpu.SemaphoreType.DMA,
    ],
    compiler_params=pltpu.CompilerParams(needs_layout_passes=False,
                                         use_tc_tiling_on_sc=True),
)(_sc_body)


def _tc_body(x_ref, reg_ref, rank_ref, regl_ref):
    x = x_ref[...]
    t = -x
    sp = jnp.maximum(t, 0.0) + jnp.log1p(jnp.exp(-jnp.abs(t)))
    rank_ref[0, 0] = jnp.sum(sp) * (1.0 / BATCH)
    regl_ref[0, 0] = jnp.sum(reg_ref[...]) * (1.0 / BATCH)


_tc_kernel = pl.pallas_call(
    _tc_body,
    out_shape=(
        jax.ShapeDtypeStruct((1, 1), jnp.float32),
        jax.ShapeDtypeStruct((1, 1), jnp.float32),
    ),
    in_specs=[
        pl.BlockSpec(memory_space=pltpu.VMEM),
        pl.BlockSpec(memory_space=pltpu.VMEM),
    ],
    out_specs=(
        pl.BlockSpec(memory_space=pltpu.SMEM),
        pl.BlockSpec(memory_space=pltpu.SMEM),
    ),
)


@jax.jit
def kernel(users, pos_items, neg_items, user_emb, item_emb):
    users3 = users.astype(jnp.float32).reshape(NW, 4, 128)
    pos3 = pos_items.astype(jnp.float32).reshape(NW, 4, 128)
    neg3 = neg_items.astype(jnp.float32).reshape(NW, 4, 128)
    uemb3 = user_emb[:NBLK * 8].reshape(NBLK, 8, DIM)
    iemb3 = item_emb[:NBLK * 8].reshape(NBLK, 8, DIM)
    x, reg_part = _sc_kernel(users3, pos3, neg3, uemb3, iemb3)
    rank, regl = _tc_kernel(x.reshape(128, 128), reg_part.reshape(32, 128))
    return (rank[0, 0], regl[0, 0])
